# baseline (speedup 1.0000x reference)
import jax
import jax.numpy as jnp
from jax.experimental import pallas as pl
from jax.experimental.pallas import tpu as pltpu

_NH = 16
_KD = 64
_VD = 64
_D = _NH * _KD


def _outer_kernel(probs_ref, vm_ref, out_ref, nn_ref):
    pp = probs_ref[...]
    p0 = jnp.transpose(pp[:, :, 0], (1, 0))
    p1 = jnp.transpose(pp[:, :, 1], (1, 0))
    vm0 = vm_ref[0]
    vm1 = vm_ref[1]
    out_ref[...] = (p0[:, :, None] * vm0[:, None, :]
                    + p1[:, :, None] * vm1[:, None, :])
    nn_ref[...] = p0


def _fused_kernel(x_ref, wk_ref, wv_ref, h_hbm, wq_ref,
                  probs_ref, vm_out, hbuf, kxs, vms, sem):
    n = pl.program_id(0)
    nb = pl.num_programs(0)
    nbuf = 3

    def h_copy(blk, slot):
        return pltpu.make_async_copy(
            h_hbm.at[:, blk, :], hbuf.at[slot], sem.at[slot])

    slot = jax.lax.rem(n, nbuf)

    @pl.when(n == 0)
    def _():
        h_copy(0, 0).start()
        h_copy(1, 1).start()

    @pl.when(n + 2 < nb)
    def _():
        h_copy(n + 2, jax.lax.rem(n + 2, nbuf)).start()

    @pl.when(n == 0)
    def _():
        wk = wk_ref[...]
        wv = wv_ref[...]
        i_idx = jax.lax.broadcasted_iota(jnp.int32, (_NH * _VD, _VD), 0)
        d_idx = jax.lax.broadcasted_iota(jnp.int32, (_NH * _VD, _VD), 1)
        e = jnp.where(i_idx % _VD == d_idx, 1.0 / _NH, 0.0)
        wvm = jax.lax.dot_general(wv, e, (((0,), (0,)), ((), ())),
                                  preferred_element_type=jnp.float32)
        for s in range(2):
            xs = x_ref[:, s, :]
            kxs[s, :, :] = jax.lax.dot_general(
                xs, wk, (((1,), (1,)), ((), ())),
                preferred_element_type=jnp.float32)
            vms[s, :, :] = jnp.dot(xs, wvm,
                                   preferred_element_type=jnp.float32)
            vm_out[s, :, :] = vms[s, :, :]

    h_copy(n, slot).wait()
    c = 1.0 / (_NH * (_KD ** 0.5))
    hb = hbuf[slot].astype(jnp.bfloat16)
    wq = wq_ref[0].astype(jnp.bfloat16)
    q = jnp.dot(hb, wq, preferred_element_type=jnp.float32)
    s0 = jnp.sum(q * kxs[0], axis=1, keepdims=True) * c
    s1 = jnp.sum(q * kxs[1], axis=1, keepdims=True) * c
    m = jnp.maximum(s0, s1)
    e0 = jnp.exp(s0 - m)
    e1 = jnp.exp(s1 - m)
    denom = e0 + e1
    p0 = e0 / denom
    p1 = e1 / denom
    probs_ref[0] = jnp.concatenate([p0, p1], axis=1)


def kernel(x, h, Wk, Wv, Wq):
    b = x.shape[0]
    hs = h.shape[2]
    nb = h.shape[1]
    is_ = x.shape[2]

    probs_t, vm = pl.pallas_call(
        _fused_kernel,
        grid=(nb,),
        in_specs=[
            pl.BlockSpec((b, 2, is_), lambda n: (0, 0, 0)),
            pl.BlockSpec((_D, is_), lambda n: (0, 0)),
            pl.BlockSpec((_NH * _VD, is_), lambda n: (0, 0)),
            pl.BlockSpec(memory_space=pl.ANY),
            pl.BlockSpec((1, hs, _D), lambda n: (n, 0, 0)),
        ],
        scratch_shapes=[
            pltpu.VMEM((3, b, hs), jnp.float32),
            pltpu.VMEM((2, b, _D), jnp.float32),
            pltpu.VMEM((2, b, _VD), jnp.float32),
            pltpu.SemaphoreType.DMA((3,)),
        ],
        out_specs=(
            pl.BlockSpec((1, b, 2), lambda n: (n, 0, 0)),
            pl.BlockSpec((2, b, _VD), lambda n: (0, 0, 0)),
        ),
        out_shape=(
            jax.ShapeDtypeStruct((nb, b, 2), jnp.float32),
            jax.ShapeDtypeStruct((2, b, _VD), jnp.float32),
        ),
    )(x, Wk, Wv, h, Wq)

    bt = 128
    inputs, not_null = pl.pallas_call(
        _outer_kernel,
        grid=(b // bt,),
        in_specs=[
            pl.BlockSpec((nb, bt, 2), lambda i: (0, i, 0)),
            pl.BlockSpec((2, bt, _VD), lambda i: (0, i, 0)),
        ],
        out_specs=(
            pl.BlockSpec((bt, nb, _VD), lambda i: (i, 0, 0)),
            pl.BlockSpec((bt, nb), lambda i: (i, 0)),
        ),
        out_shape=(
            jax.ShapeDtypeStruct((b, nb, _VD), jnp.float32),
            jax.ShapeDtypeStruct((b, nb), jnp.float32),
        ),
    )(probs_t, vm)

    mask = jnp.ones((b, nb), x.dtype)
    reg_loss = jnp.zeros((1,), x.dtype)
    return (inputs, mask, not_null, reg_loss)


# speedup vs baseline: 1.0931x; 1.0931x over previous
import jax
import jax.numpy as jnp
from jax.experimental import pallas as pl
from jax.experimental.pallas import tpu as pltpu

_NH = 16
_KD = 64
_VD = 64
_D = _NH * _KD


def _fused_kernel(x_ref, wk_ref, wv_ref, h_hbm, wq_ref,
                  out_ref, probs_ref, hbuf, kxs, vms, sem):
    n = pl.program_id(0)
    nb = pl.num_programs(0)
    nbuf = 3

    def h_copy(blk, slot):
        return pltpu.make_async_copy(
            h_hbm.at[:, blk, :], hbuf.at[slot], sem.at[slot])

    slot = jax.lax.rem(n, nbuf)

    @pl.when(n == 0)
    def _():
        h_copy(0, 0).start()
        h_copy(1, 1).start()

    @pl.when(n + 2 < nb)
    def _():
        h_copy(n + 2, jax.lax.rem(n + 2, nbuf)).start()

    @pl.when(n == 0)
    def _():
        wk = wk_ref[...]
        wv = wv_ref[...]
        i_idx = jax.lax.broadcasted_iota(jnp.int32, (_NH * _VD, _VD), 0)
        d_idx = jax.lax.broadcasted_iota(jnp.int32, (_NH * _VD, _VD), 1)
        e = jnp.where(i_idx % _VD == d_idx, 1.0 / _NH, 0.0)
        wvm = jax.lax.dot_general(wv, e, (((0,), (0,)), ((), ())),
                                  preferred_element_type=jnp.float32)
        for s in range(2):
            xs = x_ref[:, s, :]
            kxs[s, :, :] = jax.lax.dot_general(
                xs, wk, (((1,), (1,)), ((), ())),
                preferred_element_type=jnp.float32)
            vms[s, :, :] = jnp.dot(xs, wvm,
                                   preferred_element_type=jnp.float32)

    h_copy(n, slot).wait()
    c = 1.0 / (_NH * (_KD ** 0.5))
    hb = hbuf[slot].astype(jnp.bfloat16)
    wq = wq_ref[0].astype(jnp.bfloat16)
    q = jnp.dot(hb, wq, preferred_element_type=jnp.float32)
    s0 = jnp.sum(q * kxs[0], axis=1, keepdims=True) * c
    s1 = jnp.sum(q * kxs[1], axis=1, keepdims=True) * c
    m = jnp.maximum(s0, s1)
    e0 = jnp.exp(s0 - m)
    e1 = jnp.exp(s1 - m)
    denom = e0 + e1
    p0 = e0 / denom
    p1 = e1 / denom
    out_ref[0] = p0 * vms[0] + p1 * vms[1]
    probs_ref[0] = jnp.concatenate([p0, p1], axis=1)


def kernel(x, h, Wk, Wv, Wq):
    b = x.shape[0]
    hs = h.shape[2]
    nb = h.shape[1]
    is_ = x.shape[2]

    inputs_t, probs_t = pl.pallas_call(
        _fused_kernel,
        grid=(nb,),
        in_specs=[
            pl.BlockSpec((b, 2, is_), lambda n: (0, 0, 0)),
            pl.BlockSpec((_D, is_), lambda n: (0, 0)),
            pl.BlockSpec((_NH * _VD, is_), lambda n: (0, 0)),
            pl.BlockSpec(memory_space=pl.ANY),
            pl.BlockSpec((1, hs, _D), lambda n: (n, 0, 0)),
        ],
        scratch_shapes=[
            pltpu.VMEM((3, b, hs), jnp.float32),
            pltpu.VMEM((2, b, _D), jnp.float32),
            pltpu.VMEM((2, b, _VD), jnp.float32),
            pltpu.SemaphoreType.DMA((3,)),
        ],
        out_specs=(
            pl.BlockSpec((1, b, _VD), lambda n: (n, 0, 0)),
            pl.BlockSpec((1, b, 2), lambda n: (n, 0, 0)),
        ),
        out_shape=(
            jax.ShapeDtypeStruct((nb, b, _VD), jnp.float32),
            jax.ShapeDtypeStruct((nb, b, 2), jnp.float32),
        ),
    )(x, Wk, Wv, h, Wq)

    inputs = inputs_t.transpose(1, 0, 2)
    not_null = probs_t[:, :, 0].T
    mask = jnp.ones((b, nb), x.dtype)
    reg_loss = jnp.zeros((1,), x.dtype)
    return (inputs, mask, not_null, reg_loss)
